# baseline (speedup 1.0000x reference)
import functools

import jax
import jax.numpy as jnp
from jax import lax
from jax.experimental import pallas as pl
from jax.experimental.pallas import tpu as pltpu
from jax.experimental.pallas import tpu_sc as plsc

_VOCAB = 1000000
_EMBED = 64
_BATCH = 4096
_SEQ = 200
_EPS = 1e-5

_NC = 2
_NS = 16
_NW = _NC * _NS
_BPW = _BATCH // _NW
_S0 = 104
_S1 = _SEQ - _S0
_G = _EMBED // 16


def _pool_body(ids_hbm, table_hbm, pooled_hbm, idx_v, buf0, buf1, out_v,
               sem0, sem1):
    wid = lax.axis_index("s") * _NC + lax.axis_index("c")
    base = wid * _BPW
    pltpu.sync_copy(ids_hbm.at[pl.ds(base * _SEQ, _BPW * _SEQ)], idx_v)

    bufs = (buf0, buf1)
    sems = (sem0, sem1)

    def issue(b, buf, sem):
        off = b * _SEQ
        pltpu.async_copy(
            table_hbm.at[idx_v.at[pl.ds(off, _S0)]],
            buf.at[pl.ds(0, _S0)], sem)
        pltpu.async_copy(
            table_hbm.at[idx_v.at[pl.ds(off + _S0, _S1)]],
            buf.at[pl.ds(_S0, _S1)], sem)

    def drain(buf, sem):
        pltpu.make_async_copy(table_hbm.at[pl.ds(0, _SEQ)], buf, sem).wait()

    def accum(b, buf):
        def body(r, accs):
            return tuple(accs[g] + buf[r, pl.ds(g * 16, 16)]
                         for g in range(_G))
        accs = tuple(jnp.zeros((16,), jnp.float32) for _ in range(_G))
        accs = lax.fori_loop(0, _SEQ, body, accs, unroll=4)
        for g in range(_G):
            out_v[b, pl.ds(g * 16, 16)] = accs[g] * (1.0 / _SEQ)

    issue(0, buf0, sem0)

    def outer(i, carry):
        b = i * 2
        for p in range(2):
            bb = b + p

            @pl.when(bb + 1 < _BPW)
            def _():
                issue(bb + 1, bufs[1 - p], sems[1 - p])

            drain(bufs[p], sems[p])
            accum(bb, bufs[p])
        return carry

    lax.fori_loop(0, _BPW // 2, outer, 0)
    pltpu.sync_copy(out_v, pooled_hbm.at[pl.ds(base, _BPW)])


_pool = functools.partial(
    pl.kernel,
    out_type=jax.ShapeDtypeStruct((_BATCH, _EMBED), jnp.float32),
    mesh=plsc.VectorSubcoreMesh(core_axis_name="c", subcore_axis_name="s"),
    scratch_types=[
        pltpu.VMEM((_BPW * _SEQ,), jnp.int32),
        pltpu.VMEM((_SEQ, _EMBED), jnp.float32),
        pltpu.VMEM((_SEQ, _EMBED), jnp.float32),
        pltpu.VMEM((_BPW, _EMBED), jnp.float32),
        pltpu.SemaphoreType.DMA,
        pltpu.SemaphoreType.DMA,
    ],
    compiler_params=pltpu.CompilerParams(use_tc_tiling_on_sc=False),
)(_pool_body)


def _head_body(pooled_ref, wt_ref, b_ref, bng_ref, bnb_ref, lng_ref, lnb_ref,
               out_ref):
    p = jnp.dot(pooled_ref[:], wt_ref[:],
                preferred_element_type=jnp.float32) + b_ref[:]
    mu_b = jnp.mean(p, axis=0, keepdims=True)
    d = p - mu_b
    var_b = jnp.mean(d * d, axis=0, keepdims=True)
    bn = d * lax.rsqrt(var_b + _EPS) * bng_ref[:] + bnb_ref[:]
    mu_l = jnp.mean(bn, axis=1, keepdims=True)
    dl = bn - mu_l
    var_l = jnp.mean(dl * dl, axis=1, keepdims=True)
    out_ref[:] = dl * lax.rsqrt(var_l + _EPS) * lng_ref[:] + lnb_ref[:]


def kernel(input_ids, table, W, b, bn_gamma, bn_beta, ln_gamma, ln_beta):
    ids = input_ids.astype(jnp.int32).reshape(-1)
    pooled = _pool(ids, table)
    out = pl.pallas_call(
        _head_body,
        out_shape=jax.ShapeDtypeStruct((_BATCH, _EMBED), jnp.float32),
    )(pooled, W.T, b.reshape(1, -1), bn_gamma.reshape(1, -1),
      bn_beta.reshape(1, -1), ln_gamma.reshape(1, -1), ln_beta.reshape(1, -1))
    return out


# speedup vs baseline: 1.3312x; 1.3312x over previous
import functools

import jax
import jax.numpy as jnp
from jax import lax
from jax.experimental import pallas as pl
from jax.experimental.pallas import tpu as pltpu
from jax.experimental.pallas import tpu_sc as plsc

_VOCAB = 1000000
_EMBED = 64
_BATCH = 4096
_SEQ = 200
_EPS = 1e-5

_NC = 2
_NS = 16
_NW = _NC * _NS
_BPW = _BATCH // _NW
_S0 = 104
_S1 = _SEQ - _S0
_G = _EMBED // 16
_ROW = 128
_TBLK = 8192


def _tpack_body(tt_ref, out_ref):
    xt = tt_ref[...].T
    out_ref[...] = jnp.concatenate([xt, xt], axis=1)


def _tpack(tableT):
    nblk = (_VOCAB + _TBLK - 1) // _TBLK
    return pl.pallas_call(
        _tpack_body,
        grid=(nblk,),
        in_specs=[pl.BlockSpec((_EMBED, _TBLK), lambda i: (0, i))],
        out_specs=pl.BlockSpec((_TBLK, _ROW), lambda i: (i, 0)),
        out_shape=jax.ShapeDtypeStruct((_VOCAB, _ROW), jnp.float32),
    )(tableT)


def _pool_body(ids_hbm, table_hbm, pooled_hbm, idx_v, buf0, buf1, out_v,
               sem0, sem1):
    wid = lax.axis_index("s") * _NC + lax.axis_index("c")
    base = wid * _BPW
    pltpu.sync_copy(ids_hbm.at[pl.ds(base * _SEQ, _BPW * _SEQ)], idx_v)

    bufs = (buf0, buf1)
    sems = (sem0, sem1)

    def issue(b, buf, sem):
        off = b * _SEQ
        pltpu.async_copy(
            table_hbm.at[idx_v.at[pl.ds(off, _S0)]],
            buf.at[pl.ds(0, _S0)], sem)
        pltpu.async_copy(
            table_hbm.at[idx_v.at[pl.ds(off + _S0, _S1)]],
            buf.at[pl.ds(_S0, _S1)], sem)

    def drain(buf, sem):
        pltpu.make_async_copy(table_hbm.at[pl.ds(0, _SEQ)], buf, sem).wait()

    def accum(b, buf):
        def body(r, accs):
            return tuple(accs[g] + buf[r, pl.ds(g * 16, 16)]
                         for g in range(_G))
        accs = tuple(jnp.zeros((16,), jnp.float32) for _ in range(_G))
        accs = lax.fori_loop(0, _SEQ, body, accs, unroll=4)
        for g in range(_G):
            out_v[b, pl.ds(g * 16, 16)] = accs[g] * (1.0 / _SEQ)

    issue(0, buf0, sem0)

    def outer(i, carry):
        b = i * 2
        for p in range(2):
            bb = b + p

            @pl.when(bb + 1 < _BPW)
            def _():
                issue(bb + 1, bufs[1 - p], sems[1 - p])

            drain(bufs[p], sems[p])
            accum(bb, bufs[p])
        return carry

    lax.fori_loop(0, _BPW // 2, outer, 0)
    pltpu.sync_copy(out_v, pooled_hbm.at[pl.ds(base, _BPW)])


_pool = functools.partial(
    pl.kernel,
    out_type=jax.ShapeDtypeStruct((_BATCH, _EMBED), jnp.float32),
    mesh=plsc.VectorSubcoreMesh(core_axis_name="c", subcore_axis_name="s"),
    scratch_types=[
        pltpu.VMEM((_BPW * _SEQ,), jnp.int32),
        pltpu.VMEM((_SEQ, _ROW), jnp.float32),
        pltpu.VMEM((_SEQ, _ROW), jnp.float32),
        pltpu.VMEM((_BPW, _EMBED), jnp.float32),
        pltpu.SemaphoreType.DMA,
        pltpu.SemaphoreType.DMA,
    ],
    compiler_params=pltpu.CompilerParams(use_tc_tiling_on_sc=False),
)(_pool_body)


def _head_body(pooled_ref, wt_ref, b_ref, bng_ref, bnb_ref, lng_ref, lnb_ref,
               out_ref):
    p = jnp.dot(pooled_ref[:], wt_ref[:],
                preferred_element_type=jnp.float32) + b_ref[:]
    mu_b = jnp.mean(p, axis=0, keepdims=True)
    d = p - mu_b
    var_b = jnp.mean(d * d, axis=0, keepdims=True)
    bn = d * lax.rsqrt(var_b + _EPS) * bng_ref[:] + bnb_ref[:]
    mu_l = jnp.mean(bn, axis=1, keepdims=True)
    dl = bn - mu_l
    var_l = jnp.mean(dl * dl, axis=1, keepdims=True)
    out_ref[:] = dl * lax.rsqrt(var_l + _EPS) * lng_ref[:] + lnb_ref[:]


def kernel(input_ids, table, W, b, bn_gamma, bn_beta, ln_gamma, ln_beta):
    ids = input_ids.astype(jnp.int32).reshape(-1)
    tpad = _tpack(table.T)
    pooled = _pool(ids, tpad)
    out = pl.pallas_call(
        _head_body,
        out_shape=jax.ShapeDtypeStruct((_BATCH, _EMBED), jnp.float32),
    )(pooled, W.T, b.reshape(1, -1), bn_gamma.reshape(1, -1),
      bn_beta.reshape(1, -1), ln_gamma.reshape(1, -1), ln_beta.reshape(1, -1))
    return out


# speedup vs baseline: 1.4871x; 1.1171x over previous
import functools

import jax
import jax.numpy as jnp
from jax import lax
from jax.experimental import pallas as pl
from jax.experimental.pallas import tpu as pltpu
from jax.experimental.pallas import tpu_sc as plsc

_VOCAB = 1000000
_EMBED = 64
_BATCH = 4096
_SEQ = 200
_EPS = 1e-5

_NC = 2
_NS = 16
_NW = _NC * _NS
_BPW = _BATCH // _NW
_S0 = 104
_S1 = _SEQ - _S0
_G = _EMBED // 16
_ROW = 128
_TBLK = 16384
_NBUF = 3


def _tpack_body(tt_ref, out_ref):
    xt = tt_ref[...].T
    out_ref[...] = jnp.concatenate([xt, xt], axis=1)


def _tpack(tableT):
    nblk = (_VOCAB + _TBLK - 1) // _TBLK
    return pl.pallas_call(
        _tpack_body,
        grid=(nblk,),
        in_specs=[pl.BlockSpec((_EMBED, _TBLK), lambda i: (0, i))],
        out_specs=pl.BlockSpec((_TBLK, _ROW), lambda i: (i, 0)),
        out_shape=jax.ShapeDtypeStruct((_VOCAB, _ROW), jnp.float32),
    )(tableT)


def _pool_body(ids_hbm, table_hbm, pooled_hbm, idx_v, bufs, out_v, sems):
    wid = lax.axis_index("s") * _NC + lax.axis_index("c")
    base = wid * _BPW
    pltpu.sync_copy(ids_hbm.at[pl.ds(base * _SEQ, _BPW * _SEQ)], idx_v)

    def issue(b, buf, sem):
        off = b * _SEQ
        pltpu.async_copy(
            table_hbm.at[idx_v.at[pl.ds(off, _S0)]],
            buf.at[pl.ds(0, _S0)], sem)
        pltpu.async_copy(
            table_hbm.at[idx_v.at[pl.ds(off + _S0, _S1)]],
            buf.at[pl.ds(_S0, _S1)], sem)

    def drain(buf, sem):
        pltpu.make_async_copy(table_hbm.at[pl.ds(0, _SEQ)], buf, sem).wait()

    def accum(b, buf):
        def body(r, accs):
            return tuple(accs[g] + buf[r, pl.ds(g * 16, 16)]
                         for g in range(_G))
        accs = tuple(jnp.zeros((16,), jnp.float32) for _ in range(_G))
        accs = lax.fori_loop(0, _SEQ, body, accs, unroll=4)
        for g in range(_G):
            out_v[b, pl.ds(g * 16, 16)] = accs[g] * (1.0 / _SEQ)

    for p in range(_NBUF - 1):
        issue(p, bufs[p], sems[p])

    def outer(i, carry):
        b = i * _NBUF
        for p in range(_NBUF):
            bb = b + p
            nxt = bb + _NBUF - 1

            @pl.when(nxt < _BPW)
            def _():
                issue(nxt, bufs[(p + _NBUF - 1) % _NBUF],
                      sems[(p + _NBUF - 1) % _NBUF])

            drain(bufs[p], sems[p])
            accum(bb, bufs[p])
        return carry

    ntail = _BPW % _NBUF
    nloop = _BPW // _NBUF
    lax.fori_loop(0, nloop, outer, 0)
    for p in range(ntail):
        drain(bufs[p], sems[p])
        accum(nloop * _NBUF + p, bufs[p])
    pltpu.sync_copy(out_v, pooled_hbm.at[pl.ds(base, _BPW)])


_pool = functools.partial(
    pl.kernel,
    out_type=jax.ShapeDtypeStruct((_BATCH, _EMBED), jnp.float32),
    mesh=plsc.VectorSubcoreMesh(core_axis_name="c", subcore_axis_name="s"),
    scratch_types=[
        pltpu.VMEM((_BPW * _SEQ,), jnp.int32),
        tuple(pltpu.VMEM((_SEQ, _ROW), jnp.float32) for _ in range(_NBUF)),
        pltpu.VMEM((_BPW, _EMBED), jnp.float32),
        tuple(pltpu.SemaphoreType.DMA for _ in range(_NBUF)),
    ],
    compiler_params=pltpu.CompilerParams(use_tc_tiling_on_sc=False),
)(_pool_body)


def _head_body(pooled_ref, wt_ref, b_ref, bng_ref, bnb_ref, lng_ref, lnb_ref,
               out_ref):
    p = jnp.dot(pooled_ref[:], wt_ref[:],
                preferred_element_type=jnp.float32) + b_ref[:]
    mu_b = jnp.mean(p, axis=0, keepdims=True)
    d = p - mu_b
    var_b = jnp.mean(d * d, axis=0, keepdims=True)
    bn = d * lax.rsqrt(var_b + _EPS) * bng_ref[:] + bnb_ref[:]
    mu_l = jnp.mean(bn, axis=1, keepdims=True)
    dl = bn - mu_l
    var_l = jnp.mean(dl * dl, axis=1, keepdims=True)
    out_ref[:] = dl * lax.rsqrt(var_l + _EPS) * lng_ref[:] + lnb_ref[:]


def kernel(input_ids, table, W, b, bn_gamma, bn_beta, ln_gamma, ln_beta):
    ids = input_ids.astype(jnp.int32).reshape(-1)
    tpad = _tpack(table.T)
    pooled = _pool(ids, tpad)
    out = pl.pallas_call(
        _head_body,
        out_shape=jax.ShapeDtypeStruct((_BATCH, _EMBED), jnp.float32),
    )(pooled, W.T, b.reshape(1, -1), bn_gamma.reshape(1, -1),
      bn_beta.reshape(1, -1), ln_gamma.reshape(1, -1), ln_beta.reshape(1, -1))
    return out


# speedup vs baseline: 2.2817x; 1.5344x over previous
import functools

import jax
import jax.numpy as jnp
from jax import lax
from jax.experimental import pallas as pl
from jax.experimental.pallas import tpu as pltpu
from jax.experimental.pallas import tpu_sc as plsc

_VOCAB = 1000000
_EMBED = 64
_BATCH = 4096
_SEQ = 200
_EPS = 1e-5

_NC = 2
_NS = 16
_NW = _NC * _NS
_BPW = _BATCH // _NW
_S0 = 104
_S1 = _SEQ - _S0
_G = _EMBED // 16
_ROW = 128
_TBLK = 16384
_FOLD = 507904
_NBUF = 6


def _tpack_body(lo_ref, hi_ref, out_ref):
    out_ref[...] = jnp.concatenate([lo_ref[...].T, hi_ref[...].T], axis=1)


def _tpack(tableT):
    nblk = _FOLD // _TBLK
    return pl.pallas_call(
        _tpack_body,
        grid=(nblk,),
        in_specs=[
            pl.BlockSpec((_EMBED, _TBLK), lambda i: (0, i)),
            pl.BlockSpec((_EMBED, _TBLK), lambda i: (0, i + _FOLD // _TBLK)),
        ],
        out_specs=pl.BlockSpec((_TBLK, _ROW), lambda i: (i, 0)),
        out_shape=jax.ShapeDtypeStruct((_FOLD, _ROW), jnp.float32),
    )(tableT, tableT)


def _pool_body(ids_hbm, table_hbm, pooled_hbm, idx_v, bufs, out_v, sems):
    wid = lax.axis_index("s") * _NC + lax.axis_index("c")
    base = wid * _BPW
    pltpu.sync_copy(ids_hbm.at[pl.ds(base * _SEQ, _BPW * _SEQ)], idx_v)

    def issue(b, buf, sem):
        off = b * _SEQ
        pltpu.async_copy(
            table_hbm.at[idx_v.at[pl.ds(off, _S0)]],
            buf.at[pl.ds(0, _S0)], sem)
        pltpu.async_copy(
            table_hbm.at[idx_v.at[pl.ds(off + _S0, _S1)]],
            buf.at[pl.ds(_S0, _S1)], sem)

    def drain(buf, sem):
        pltpu.make_async_copy(table_hbm.at[pl.ds(0, _SEQ)], buf, sem).wait()

    def accum(b, buf):
        def body(r, accs):
            return tuple(accs[g] + buf[r, pl.ds(g * 16, 16)]
                         for g in range(_G))
        accs = tuple(jnp.zeros((16,), jnp.float32) for _ in range(_G))
        accs = lax.fori_loop(0, _SEQ, body, accs, unroll=4)
        for g in range(_G):
            out_v[b, pl.ds(g * 16, 16)] = accs[g] * (1.0 / _SEQ)

    for p in range(_NBUF - 1):
        issue(p, bufs[p], sems[p])

    def outer(i, carry):
        b = i * _NBUF
        for p in range(_NBUF):
            bb = b + p
            nxt = bb + _NBUF - 1

            @pl.when(nxt < _BPW)
            def _():
                issue(nxt, bufs[(p + _NBUF - 1) % _NBUF],
                      sems[(p + _NBUF - 1) % _NBUF])

            drain(bufs[p], sems[p])
            accum(bb, bufs[p])
        return carry

    ntail = _BPW % _NBUF
    nloop = _BPW // _NBUF
    lax.fori_loop(0, nloop, outer, 0)
    for p in range(ntail):
        drain(bufs[p], sems[p])
        accum(nloop * _NBUF + p, bufs[p])
    pltpu.sync_copy(out_v, pooled_hbm.at[pl.ds(base, _BPW)])


_pool = functools.partial(
    pl.kernel,
    out_type=jax.ShapeDtypeStruct((_BATCH, _EMBED), jnp.float32),
    mesh=plsc.VectorSubcoreMesh(core_axis_name="c", subcore_axis_name="s"),
    scratch_types=[
        pltpu.VMEM((_BPW * _SEQ,), jnp.int32),
        tuple(pltpu.VMEM((_SEQ, _EMBED), jnp.float32) for _ in range(_NBUF)),
        pltpu.VMEM((_BPW, _EMBED), jnp.float32),
        tuple(pltpu.SemaphoreType.DMA for _ in range(_NBUF)),
    ],
    compiler_params=pltpu.CompilerParams(use_tc_tiling_on_sc=False),
)(_pool_body)


def _head_body(pooled_ref, wt_ref, b_ref, bng_ref, bnb_ref, lng_ref, lnb_ref,
               out_ref):
    p = jnp.dot(pooled_ref[:], wt_ref[:],
                preferred_element_type=jnp.float32) + b_ref[:]
    mu_b = jnp.mean(p, axis=0, keepdims=True)
    d = p - mu_b
    var_b = jnp.mean(d * d, axis=0, keepdims=True)
    bn = d * lax.rsqrt(var_b + _EPS) * bng_ref[:] + bnb_ref[:]
    mu_l = jnp.mean(bn, axis=1, keepdims=True)
    dl = bn - mu_l
    var_l = jnp.mean(dl * dl, axis=1, keepdims=True)
    out_ref[:] = dl * lax.rsqrt(var_l + _EPS) * lng_ref[:] + lnb_ref[:]


def kernel(input_ids, table, W, b, bn_gamma, bn_beta, ln_gamma, ln_beta):
    v = input_ids.astype(jnp.int32)
    hi = (v >= _FOLD).astype(jnp.int32)
    g2 = ((v - hi * _FOLD) * 2 + hi).reshape(-1)
    tpad = _tpack(table.T).reshape(2 * _FOLD, _EMBED)
    pooled = _pool(g2, tpad)
    out = pl.pallas_call(
        _head_body,
        out_shape=jax.ShapeDtypeStruct((_BATCH, _EMBED), jnp.float32),
    )(pooled, W.T, b.reshape(1, -1), bn_gamma.reshape(1, -1),
      bn_beta.reshape(1, -1), ln_gamma.reshape(1, -1), ln_beta.reshape(1, -1))
    return out


# speedup vs baseline: 2.6861x; 1.1772x over previous
import functools

import jax
import jax.numpy as jnp
from jax import lax
from jax.experimental import pallas as pl
from jax.experimental.pallas import tpu as pltpu
from jax.experimental.pallas import tpu_sc as plsc

_VOCAB = 1000000
_EMBED = 64
_BATCH = 4096
_SEQ = 200
_EPS = 1e-5

_NC = 2
_NS = 16
_NW = _NC * _NS
_BPW = _BATCH // _NW
_S0 = 104
_S1 = _SEQ - _S0
_G = _EMBED // 16
_ROW = 128
_TBLK = 16384
_FOLD = 507904
_NBUF = 6


def _tpack_body(lo_ref, hi_ref, out_ref):
    out_ref[...] = jnp.concatenate([lo_ref[...], hi_ref[...]], axis=0).T


def _tpack(tableT):
    nblk = _FOLD // _TBLK
    return pl.pallas_call(
        _tpack_body,
        grid=(nblk,),
        in_specs=[
            pl.BlockSpec((_EMBED, _TBLK), lambda i: (0, i)),
            pl.BlockSpec((_EMBED, _TBLK), lambda i: (0, i + _FOLD // _TBLK)),
        ],
        out_specs=pl.BlockSpec((_TBLK, _ROW), lambda i: (i, 0)),
        out_shape=jax.ShapeDtypeStruct((_FOLD, _ROW), jnp.float32),
        compiler_params=pltpu.CompilerParams(
            fuse_transposed_lhs_in_matmul=True),
    )(tableT, tableT)


def _pool_body(ids_hbm, table_hbm, pooled_hbm, idx_v, bufs, out_v, sems):
    wid = lax.axis_index("s") * _NC + lax.axis_index("c")
    base = wid * _BPW
    pltpu.sync_copy(ids_hbm.at[pl.ds(base * _SEQ, _BPW * _SEQ)], idx_v)

    def issue(b, buf, sem):
        off = b * _SEQ
        pltpu.async_copy(
            table_hbm.at[idx_v.at[pl.ds(off, _S0)]],
            buf.at[pl.ds(0, _S0)], sem)
        pltpu.async_copy(
            table_hbm.at[idx_v.at[pl.ds(off + _S0, _S1)]],
            buf.at[pl.ds(_S0, _S1)], sem)

    def drain(buf, sem):
        pltpu.make_async_copy(table_hbm.at[pl.ds(0, _SEQ)], buf, sem).wait()

    def accum(b, buf):
        def body(r, accs):
            return tuple(accs[g] + buf[r, pl.ds(g * 16, 16)]
                         for g in range(_G))
        accs = tuple(jnp.zeros((16,), jnp.float32) for _ in range(_G))
        accs = lax.fori_loop(0, _SEQ, body, accs, unroll=4)
        for g in range(_G):
            out_v[b, pl.ds(g * 16, 16)] = accs[g] * (1.0 / _SEQ)

    for p in range(_NBUF - 1):
        issue(p, bufs[p], sems[p])

    def outer(i, carry):
        b = i * _NBUF
        for p in range(_NBUF):
            bb = b + p
            nxt = bb + _NBUF - 1

            @pl.when(nxt < _BPW)
            def _():
                issue(nxt, bufs[(p + _NBUF - 1) % _NBUF],
                      sems[(p + _NBUF - 1) % _NBUF])

            drain(bufs[p], sems[p])
            accum(bb, bufs[p])
        return carry

    ntail = _BPW % _NBUF
    nloop = _BPW // _NBUF
    lax.fori_loop(0, nloop, outer, 0)
    for p in range(ntail):
        drain(bufs[p], sems[p])
        accum(nloop * _NBUF + p, bufs[p])
    pltpu.sync_copy(out_v, pooled_hbm.at[pl.ds(base, _BPW)])


_pool = functools.partial(
    pl.kernel,
    out_type=jax.ShapeDtypeStruct((_BATCH, _EMBED), jnp.float32),
    mesh=plsc.VectorSubcoreMesh(core_axis_name="c", subcore_axis_name="s"),
    scratch_types=[
        pltpu.VMEM((_BPW * _SEQ,), jnp.int32),
        tuple(pltpu.VMEM((_SEQ, _EMBED), jnp.float32) for _ in range(_NBUF)),
        pltpu.VMEM((_BPW, _EMBED), jnp.float32),
        tuple(pltpu.SemaphoreType.DMA for _ in range(_NBUF)),
    ],
    compiler_params=pltpu.CompilerParams(use_tc_tiling_on_sc=False),
)(_pool_body)


def _head_body(pooled_ref, wt_ref, b_ref, bng_ref, bnb_ref, lng_ref, lnb_ref,
               out_ref):
    p = jnp.dot(pooled_ref[:], wt_ref[:],
                preferred_element_type=jnp.float32) + b_ref[:]
    mu_b = jnp.mean(p, axis=0, keepdims=True)
    d = p - mu_b
    var_b = jnp.mean(d * d, axis=0, keepdims=True)
    bn = d * lax.rsqrt(var_b + _EPS) * bng_ref[:] + bnb_ref[:]
    mu_l = jnp.mean(bn, axis=1, keepdims=True)
    dl = bn - mu_l
    var_l = jnp.mean(dl * dl, axis=1, keepdims=True)
    out_ref[:] = dl * lax.rsqrt(var_l + _EPS) * lng_ref[:] + lnb_ref[:]


def kernel(input_ids, table, W, b, bn_gamma, bn_beta, ln_gamma, ln_beta):
    v = input_ids.astype(jnp.int32)
    hi = (v >= _FOLD).astype(jnp.int32)
    g2 = ((v - hi * _FOLD) * 2 + hi).reshape(-1)
    tpad = _tpack(table.T).reshape(2 * _FOLD, _EMBED)
    pooled = _pool(g2, tpad)
    out = pl.pallas_call(
        _head_body,
        out_shape=jax.ShapeDtypeStruct((_BATCH, _EMBED), jnp.float32),
    )(pooled, W.T, b.reshape(1, -1), bn_gamma.reshape(1, -1),
      bn_beta.reshape(1, -1), ln_gamma.reshape(1, -1), ln_beta.reshape(1, -1))
    return out


# speedup vs baseline: 2.7729x; 1.0323x over previous
import functools

import jax
import jax.numpy as jnp
from jax import lax
from jax.experimental import pallas as pl
from jax.experimental.pallas import tpu as pltpu
from jax.experimental.pallas import tpu_sc as plsc

_VOCAB = 1000000
_EMBED = 64
_BATCH = 4096
_SEQ = 200
_EPS = 1e-5

_NC = 2
_NS = 16
_NW = _NC * _NS
_BPW = _BATCH // _NW
_S0 = 104
_S1 = _SEQ - _S0
_G = _EMBED // 16
_ROW = 128
_TBLK = 16384
_FOLD = 507904
_NBUF = 6


def _tpack_body(lo_ref, hi_ref, out_ref):
    out_ref[...] = jnp.concatenate([lo_ref[...], hi_ref[...]], axis=0).T


def _tpack(tableT):
    nblk = _FOLD // _TBLK
    return pl.pallas_call(
        _tpack_body,
        grid=(nblk,),
        in_specs=[
            pl.BlockSpec((_EMBED, _TBLK), lambda i: (0, i)),
            pl.BlockSpec((_EMBED, _TBLK), lambda i: (0, i + _FOLD // _TBLK)),
        ],
        out_specs=pl.BlockSpec((_TBLK, _ROW), lambda i: (i, 0)),
        out_shape=jax.ShapeDtypeStruct((_FOLD, _ROW), jnp.float32),
        compiler_params=pltpu.CompilerParams(
            fuse_transposed_lhs_in_matmul=True),
    )(tableT, tableT)


def _gidx_body(idsT_hbm, g2_hbm, slab_v, out_v):
    wid = lax.axis_index("s") * _NC + lax.axis_index("c")
    base = wid * _BPW
    pltpu.sync_copy(idsT_hbm.at[:, pl.ds(base, _BPW)], slab_v)
    lane = lax.iota(jnp.int32, 16)
    zero16 = lane * 0

    def xform(v):
        return v * 2 - jnp.where(v >= _FOLD, 2 * _FOLD - 1, 0)

    nfull = _SEQ // 16

    def col(b, carry):
        def chunk(k, c2):
            vals = plsc.load_gather(slab_v, [k * 16 + lane, b + zero16])
            plsc.store_scatter(out_v, [b * _SEQ + k * 16 + lane], xform(vals))
            return c2
        return lax.fori_loop(0, nfull, chunk, carry)

    lax.fori_loop(0, _BPW, col, 0)

    half = lane // 8
    low8 = lane % 8

    def tail(bp, carry):
        b2 = bp * 2
        vals = plsc.load_gather(slab_v, [nfull * 16 + low8, b2 + half])
        dst = (b2 + half) * _SEQ + nfull * 16 + low8
        plsc.store_scatter(out_v, [dst], xform(vals))
        return carry

    lax.fori_loop(0, _BPW // 2, tail, 0)
    pltpu.sync_copy(out_v, g2_hbm.at[pl.ds(base * _SEQ, _BPW * _SEQ)])


_gidx = functools.partial(
    pl.kernel,
    out_type=jax.ShapeDtypeStruct((_BATCH * _SEQ,), jnp.int32),
    mesh=plsc.VectorSubcoreMesh(core_axis_name="c", subcore_axis_name="s"),
    scratch_types=[
        pltpu.VMEM((_SEQ, _BPW), jnp.int32),
        pltpu.VMEM((_BPW * _SEQ,), jnp.int32),
    ],
    compiler_params=pltpu.CompilerParams(use_tc_tiling_on_sc=False,
                                         needs_layout_passes=False),
)(_gidx_body)


def _pool_body(ids_hbm, table_hbm, pooled_hbm, idx_v, bufs, out_v, sems):
    wid = lax.axis_index("s") * _NC + lax.axis_index("c")
    base = wid * _BPW
    pltpu.sync_copy(ids_hbm.at[pl.ds(base * _SEQ, _BPW * _SEQ)], idx_v)

    def issue(b, buf, sem):
        off = b * _SEQ
        pltpu.async_copy(
            table_hbm.at[idx_v.at[pl.ds(off, _S0)]],
            buf.at[pl.ds(0, _S0)], sem)
        pltpu.async_copy(
            table_hbm.at[idx_v.at[pl.ds(off + _S0, _S1)]],
            buf.at[pl.ds(_S0, _S1)], sem)

    def drain(buf, sem):
        pltpu.make_async_copy(table_hbm.at[pl.ds(0, _SEQ)], buf, sem).wait()

    def accum(b, buf):
        def body(r, accs):
            return tuple(accs[g] + buf[r, pl.ds(g * 16, 16)]
                         for g in range(_G))
        accs = tuple(jnp.zeros((16,), jnp.float32) for _ in range(_G))
        accs = lax.fori_loop(0, _SEQ, body, accs, unroll=4)
        for g in range(_G):
            out_v[b, pl.ds(g * 16, 16)] = accs[g] * (1.0 / _SEQ)

    for p in range(_NBUF - 1):
        issue(p, bufs[p], sems[p])

    def outer(i, carry):
        b = i * _NBUF
        for p in range(_NBUF):
            bb = b + p
            nxt = bb + _NBUF - 1

            @pl.when(nxt < _BPW)
            def _():
                issue(nxt, bufs[(p + _NBUF - 1) % _NBUF],
                      sems[(p + _NBUF - 1) % _NBUF])

            drain(bufs[p], sems[p])
            accum(bb, bufs[p])
        return carry

    ntail = _BPW % _NBUF
    nloop = _BPW // _NBUF
    lax.fori_loop(0, nloop, outer, 0)
    for p in range(ntail):
        drain(bufs[p], sems[p])
        accum(nloop * _NBUF + p, bufs[p])
    pltpu.sync_copy(out_v, pooled_hbm.at[pl.ds(base, _BPW)])


_pool = functools.partial(
    pl.kernel,
    out_type=jax.ShapeDtypeStruct((_BATCH, _EMBED), jnp.float32),
    mesh=plsc.VectorSubcoreMesh(core_axis_name="c", subcore_axis_name="s"),
    scratch_types=[
        pltpu.VMEM((_BPW * _SEQ,), jnp.int32),
        tuple(pltpu.VMEM((_SEQ, _EMBED), jnp.float32) for _ in range(_NBUF)),
        pltpu.VMEM((_BPW, _EMBED), jnp.float32),
        tuple(pltpu.SemaphoreType.DMA for _ in range(_NBUF)),
    ],
    compiler_params=pltpu.CompilerParams(use_tc_tiling_on_sc=False),
)(_pool_body)


def _head_body(pooled_ref, wt_ref, b_ref, bng_ref, bnb_ref, lng_ref, lnb_ref,
               out_ref):
    p = jnp.dot(pooled_ref[:], wt_ref[:],
                preferred_element_type=jnp.float32) + b_ref[:]
    mu_b = jnp.mean(p, axis=0, keepdims=True)
    d = p - mu_b
    var_b = jnp.mean(d * d, axis=0, keepdims=True)
    bn = d * lax.rsqrt(var_b + _EPS) * bng_ref[:] + bnb_ref[:]
    mu_l = jnp.mean(bn, axis=1, keepdims=True)
    dl = bn - mu_l
    var_l = jnp.mean(dl * dl, axis=1, keepdims=True)
    out_ref[:] = dl * lax.rsqrt(var_l + _EPS) * lng_ref[:] + lnb_ref[:]


def kernel(input_ids, table, W, b, bn_gamma, bn_beta, ln_gamma, ln_beta):
    g2 = _gidx(input_ids.astype(jnp.int32).T)
    tpad = _tpack(table.T).reshape(2 * _FOLD, _EMBED)
    pooled = _pool(g2, tpad)
    out = pl.pallas_call(
        _head_body,
        out_shape=jax.ShapeDtypeStruct((_BATCH, _EMBED), jnp.float32),
    )(pooled, W.T, b.reshape(1, -1), bn_gamma.reshape(1, -1),
      bn_beta.reshape(1, -1), ln_gamma.reshape(1, -1), ln_beta.reshape(1, -1))
    return out
